# baseline (speedup 1.0000x reference)
import functools

import jax
import jax.numpy as jnp
from jax import lax
from jax.experimental import pallas as pl
from jax.experimental.pallas import tpu as pltpu
from jax.experimental.pallas import tpu_sc as plsc


def _build_sc_kernel(B, L, LP, D, rows_per_w, nc):
    mesh = plsc.VectorSubcoreMesh(core_axis_name="c", subcore_axis_name="s")
    LG = ((L + 7) // 8) * 8
    NT = rows_per_w * LP
    WCH = 128
    n_lchunk = LP // 16
    n_half = D // 2 // 16

    @functools.partial(
        pl.kernel,
        mesh=mesh,
        out_type=jax.ShapeDtypeStruct((B, D), jnp.float32),
        compiler_params=pltpu.CompilerParams(needs_layout_passes=False),
        scratch_types=[
            pltpu.VMEM((NT,), jnp.int32),
            pltpu.VMEM((NT,), jnp.float32),
            pltpu.VMEM((NT,), jnp.int32),
            pltpu.VMEM((2, LG, D), jnp.float32),
            pltpu.VMEM((2, D), jnp.float32),
            pltpu.VMEM((16,), jnp.float32),
            pltpu.SemaphoreType.DMA,
            pltpu.SemaphoreType.DMA,
            pltpu.SemaphoreType.DMA,
        ],
    )
    def k(table, idsf, maskf, wvec, out, ids_v, w_v, mask_v, rows_v, out_v,
          red_v, sem_g, sem_o, sem_w):
        wid = lax.axis_index("s") * nc + lax.axis_index("c")
        base = wid * rows_per_w
        tbase = base * LP

        pltpu.sync_copy(idsf.at[pl.ds(tbase, NT)], ids_v)
        pltpu.sync_copy(maskf.at[pl.ds(tbase, NT)], mask_v)

        wcps = []
        for c in range(NT // WCH):
            sl = pl.ds(c * WCH, WCH)
            wcps.append(pltpu.async_copy(wvec.at[ids_v.at[sl]], w_v.at[sl], sem_w))

        def start_gather(i, t):
            idx = ids_v.at[pl.ds(i * LP, LG)]
            return pltpu.async_copy(table.at[idx], rows_v.at[t], sem_g)

        start_gather(0, 0)
        start_gather(1, 1)

        for cp in wcps:
            cp.wait()

        lanes = lax.iota(jnp.int32, 16)

        def norm_body(r, carry):
            off = r * LP
            wms = []
            total = jnp.zeros((16,), jnp.float32)
            for c in range(n_lchunk):
                sl = pl.ds(off + c * 16, 16)
                wm = w_v[sl] * mask_v[sl].astype(jnp.float32)
                wms.append(wm)
                total = total + wm
            for s in (8, 4, 2, 1):
                red_v[...] = total
                total = total + plsc.load_gather(red_v, [lanes ^ s])
            inv = jnp.float32(1.0) / total
            for c in range(n_lchunk):
                w_v[pl.ds(off + c * 16, 16)] = wms[c] * inv
            return carry

        lax.fori_loop(0, rows_per_w, norm_body, 0)

        def pair_body(j, carry):
            for t in range(2):
                i = 2 * j + t
                roff = i * LP

                @pl.when(i >= 2)
                def _():
                    pltpu.make_async_copy(out.at[base], out_v.at[t], sem_o).wait()

                pltpu.make_async_copy(
                    table.at[pl.ds(0, LG)], rows_v.at[t], sem_g
                ).wait()

                for h in range(2):
                    hoff = h * n_half * 16

                    def acc_body(l, accs):
                        wl = plsc.load_gather(w_v, [jnp.full((16,), roff + l, jnp.int32)])
                        return tuple(
                            accs[c] + wl * rows_v[t, l, pl.ds(hoff + c * 16, 16)]
                            for c in range(n_half)
                        )

                    accs = lax.fori_loop(
                        0, L, acc_body,
                        tuple(jnp.zeros((16,), jnp.float32) for _ in range(n_half)),
                    )
                    for c in range(n_half):
                        out_v[t, pl.ds(hoff + c * 16, 16)] = accs[c]

                pltpu.async_copy(out_v.at[t], out.at[base + i], sem_o)

                @pl.when(i + 2 < rows_per_w)
                def _():
                    idx = ids_v.at[pl.ds(roff + 2 * LP, LG)]
                    pltpu.async_copy(table.at[idx], rows_v.at[t], sem_g)
            return carry

        lax.fori_loop(0, rows_per_w // 2, pair_body, 0)

        for t in range(2):
            pltpu.make_async_copy(out.at[base], out_v.at[t], sem_o).wait()

    return k


def kernel(input_ids, attention_mask, tok_embs, tok_embs_weights):
    B, L = input_ids.shape
    V, D = tok_embs.shape
    info = plsc.get_sparse_core_info()
    nw = info.num_cores * info.num_subcores
    assert B % (2 * nw) == 0 and D % 32 == 0
    LP = ((L + 15) // 16) * 16
    ids_p = jnp.pad(input_ids.astype(jnp.int32), ((0, 0), (0, LP - L)))
    mask_p = jnp.pad(attention_mask.astype(jnp.int32), ((0, 0), (0, LP - L)))
    k = _build_sc_kernel(B, L, LP, D, B // nw, info.num_cores)
    return k(tok_embs, ids_p.reshape(B * LP), mask_p.reshape(B * LP),
             tok_embs_weights)


# speedup vs baseline: 1.2117x; 1.2117x over previous
import functools

import jax
import jax.numpy as jnp
from jax import lax
from jax.experimental import pallas as pl
from jax.experimental.pallas import tpu as pltpu
from jax.experimental.pallas import tpu_sc as plsc


def _build_sc_kernel(B, L, LP, D, rows_per_w, nc):
    mesh = plsc.VectorSubcoreMesh(core_axis_name="c", subcore_axis_name="s")
    LG = ((L + 7) // 8) * 8
    n_lchunk = LP // 16
    n_half = D // 2 // 16

    @functools.partial(
        pl.kernel,
        mesh=mesh,
        out_type=jax.ShapeDtypeStruct((B, D), jnp.float32),
        compiler_params=pltpu.CompilerParams(needs_layout_passes=False),
        scratch_types=[
            pltpu.VMEM((2, LP), jnp.int32),
            pltpu.VMEM((2, LP), jnp.float32),
            pltpu.VMEM((2, LP), jnp.int32),
            pltpu.VMEM((2, LG, D), jnp.float32),
            pltpu.VMEM((2, D), jnp.float32),
            pltpu.VMEM((16,), jnp.float32),
            pltpu.SemaphoreType.DMA,
            pltpu.SemaphoreType.DMA,
            pltpu.SemaphoreType.DMA,
        ],
    )
    def k(table, ids, mask, wvec, out, ids_v, w_v, mask_v, rows_v, out_v,
          red_v, sem_g, sem_w, sem_o):
        wid = lax.axis_index("s") * nc + lax.axis_index("c")
        base = wid * rows_per_w

        zf16 = jnp.zeros((16,), jnp.float32)
        for t in range(2):
            for c in range(n_lchunk):
                w_v[t, pl.ds(c * 16, 16)] = zf16

        def stage_and_launch(i, t):
            pltpu.sync_copy(ids.at[base + i], ids_v.at[t])
            pltpu.sync_copy(mask.at[base + i], mask_v.at[t])
            idx = ids_v.at[t, pl.ds(0, LG)]
            pltpu.async_copy(wvec.at[idx], w_v.at[t, pl.ds(0, LG)], sem_w)
            pltpu.async_copy(table.at[idx], rows_v.at[t], sem_g)

        def wait_row(t):
            pltpu.make_async_copy(
                wvec.at[pl.ds(0, LG)], w_v.at[t, pl.ds(0, LG)], sem_w
            ).wait()
            pltpu.make_async_copy(
                table.at[pl.ds(0, LG)], rows_v.at[t], sem_g
            ).wait()

        stage_and_launch(0, 0)
        lanes = lax.iota(jnp.int32, 16)

        def pair_body(j, carry):
            for t in range(2):
                i = 2 * j + t

                @pl.when(i + 1 < rows_per_w)
                def _():
                    stage_and_launch(i + 1, 1 - t)

                wait_row(t)

                @pl.when(i >= 2)
                def _():
                    pltpu.make_async_copy(out.at[base], out_v.at[t], sem_o).wait()

                wms = []
                total = jnp.zeros((16,), jnp.float32)
                for c in range(n_lchunk):
                    sl = pl.ds(c * 16, 16)
                    wm = w_v[t, sl] * mask_v[t, sl].astype(jnp.float32)
                    wms.append(wm)
                    total = total + wm
                for s in (8, 4, 2, 1):
                    red_v[...] = total
                    total = total + plsc.load_gather(red_v, [lanes ^ s])
                inv = jnp.float32(1.0) / total
                for c in range(n_lchunk):
                    w_v[t, pl.ds(c * 16, 16)] = wms[c] * inv

                for h in range(2):
                    hoff = h * n_half * 16

                    def acc_body(l, accs):
                        wl = plsc.load_gather(
                            w_v,
                            [jnp.full((16,), t, jnp.int32),
                             jnp.full((16,), l, jnp.int32)],
                        )
                        return tuple(
                            accs[c] + wl * rows_v[t, l, pl.ds(hoff + c * 16, 16)]
                            for c in range(n_half)
                        )

                    accs = lax.fori_loop(
                        0, L, acc_body, tuple(zf16 for _ in range(n_half))
                    )
                    for c in range(n_half):
                        out_v[t, pl.ds(hoff + c * 16, 16)] = accs[c]

                pltpu.async_copy(out_v.at[t], out.at[base + i], sem_o)
            return carry

        lax.fori_loop(0, rows_per_w // 2, pair_body, 0)

        for t in range(2):
            pltpu.make_async_copy(out.at[base], out_v.at[t], sem_o).wait()

    return k


def kernel(input_ids, attention_mask, tok_embs, tok_embs_weights):
    B, L = input_ids.shape
    V, D = tok_embs.shape
    info = plsc.get_sparse_core_info()
    nw = info.num_cores * info.num_subcores
    assert B % (2 * nw) == 0 and D % 32 == 0
    LP = ((L + 15) // 16) * 16
    ids_p = jnp.pad(input_ids.astype(jnp.int32), ((0, 0), (0, LP - L)))
    mask_p = jnp.pad(attention_mask.astype(jnp.int32), ((0, 0), (0, LP - L)))
    k = _build_sc_kernel(B, L, LP, D, B // nw, info.num_cores)
    return k(tok_embs, ids_p, mask_p, tok_embs_weights)
